# baseline (device time: 119476 ns/iter reference)
import jax
import jax.numpy as jnp
from jax import lax
from jax.experimental import pallas as pl
from jax.experimental.pallas import tpu as pltpu

N_DEV = 4
SQ = 1024
D = 1024
H = 8
DH = 128
BLK = 64
SCALE = 0.08838834764831843


def kernel(x, Wq, K_ext, V_ext, Wo):
    xf = x.reshape(SQ, D)
    kf = K_ext.reshape(K_ext.shape[1], D)
    vf = V_ext.reshape(V_ext.shape[1], D)

    def body(x_ref, wq_ref, k_ref, v_ref, wo_ref, out_ref,
             kv_ref, send_sems, recv_sem):
        my = lax.axis_index("i")

        barrier = pltpu.get_barrier_semaphore()
        for d in range(N_DEV):
            @pl.when(my != d)
            def _(d=d):
                pl.semaphore_signal(
                    barrier, inc=1,
                    device_id=(d,), device_id_type=pl.DeviceIdType.MESH,
                )
        pl.semaphore_wait(barrier, N_DEV - 1)

        def kv_rdma(target, sem_idx):
            return pltpu.make_async_remote_copy(
                src_ref=kv_ref, dst_ref=kv_ref,
                send_sem=send_sems.at[sem_idx], recv_sem=recv_sem,
                device_id=(target,), device_id_type=pl.DeviceIdType.MESH,
            )

        @pl.when(my == 0)
        def _():
            kv_ref[0] = k_ref[...].astype(jnp.bfloat16)
            kv_ref[1] = v_ref[...].astype(jnp.bfloat16)
            kv_rdma(1, 0).start()
            kv_rdma(3, 1).start()

        q = jnp.dot(
            x_ref[...].astype(jnp.bfloat16),
            wq_ref[...].astype(jnp.bfloat16),
            preferred_element_type=jnp.float32,
        ).astype(jnp.bfloat16)

        @pl.when(my == 1)
        def _():
            kv_rdma(0, 0).wait_recv()
            kv_rdma(2, 0).start()

        @pl.when((my == 2) | (my == 3))
        def _():
            kv_rdma(0, 0).wait_recv()

        kk = kv_ref[0]
        vv = kv_ref[1]

        rb = lax.broadcasted_iota(jnp.int32, (SQ, SQ), 0) // BLK
        cb = lax.broadcasted_iota(jnp.int32, (SQ, SQ), 1) // BLK
        mask = cb <= rb

        outs = []
        for h in range(H):
            hs = slice(h * DH, (h + 1) * DH)
            s = lax.dot_general(
                q[:, hs], kk[:, hs],
                (((1,), (1,)), ((), ())),
                preferred_element_type=jnp.float32,
            ) * SCALE
            s = jnp.where(mask, s, -1e9)
            m = jnp.max(s, axis=1, keepdims=True)
            w = jnp.exp(s - m)
            w = w / jnp.sum(w, axis=1, keepdims=True)
            outs.append(
                lax.dot_general(
                    w.astype(jnp.bfloat16), vv[:, hs],
                    (((1,), (0,)), ((), ())),
                    preferred_element_type=jnp.float32,
                )
            )
        ctx = jnp.concatenate(outs, axis=1).astype(jnp.bfloat16)
        out_ref[...] = jnp.dot(
            ctx, wo_ref[...].astype(jnp.bfloat16),
            preferred_element_type=jnp.float32,
        )

        @pl.when(my == 0)
        def _():
            kv_rdma(1, 0).wait_send()
            kv_rdma(3, 1).wait_send()

        @pl.when(my == 1)
        def _():
            kv_rdma(2, 0).wait_send()

    out2d = pl.pallas_call(
        body,
        out_shape=jax.ShapeDtypeStruct((SQ, D), jnp.float32),
        in_specs=[pl.BlockSpec(memory_space=pltpu.VMEM)] * 5,
        out_specs=pl.BlockSpec(memory_space=pltpu.VMEM),
        scratch_shapes=[
            pltpu.VMEM((2, SQ, D), jnp.bfloat16),
            pltpu.SemaphoreType.DMA((2,)),
            pltpu.SemaphoreType.DMA,
        ],
        compiler_params=pltpu.CompilerParams(collective_id=0),
    )(xf, Wq, kf, vf, Wo)
    return out2d.reshape(1, SQ, D)


# device time: 60214 ns/iter; 1.9842x vs baseline; 1.9842x over previous
import jax
import jax.numpy as jnp
from jax import lax
from jax.experimental import pallas as pl
from jax.experimental.pallas import tpu as pltpu

N_DEV = 4
SQ = 1024
D = 1024
H = 8
DH = 128
BLK = 64
SCALE = 0.08838834764831843

NC = 16
CHK = SQ // NC
R = list(range(0, NC // 2))
L = list(range(NC // 2, NC))


def kernel(x, Wq, K_ext, V_ext, Wo):
    xf = x.reshape(SQ, D)
    kf = K_ext.reshape(K_ext.shape[1], D)
    vf = V_ext.reshape(V_ext.shape[1], D)

    def body(x_ref, wq_ref, k_ref, v_ref, wo_ref, out_ref,
             kv_ref, send_sems, recv_sems):
        my = lax.axis_index("i")

        barrier = pltpu.get_barrier_semaphore()
        for d in range(N_DEV):
            @pl.when(my != d)
            def _(d=d):
                pl.semaphore_signal(
                    barrier, inc=1,
                    device_id=(d,), device_id_type=pl.DeviceIdType.MESH,
                )
        pl.semaphore_wait(barrier, N_DEV - 1)

        def chunk_rdma(c, target):
            return pltpu.make_async_remote_copy(
                src_ref=kv_ref.at[c], dst_ref=kv_ref.at[c],
                send_sem=send_sems.at[c], recv_sem=recv_sems.at[c],
                device_id=(target,), device_id_type=pl.DeviceIdType.MESH,
            )

        @pl.when(my == 0)
        def _():
            for c in range(NC):
                kv_ref[c, 0] = k_ref[c * CHK:(c + 1) * CHK, :].astype(
                    jnp.bfloat16)
                kv_ref[c, 1] = v_ref[c * CHK:(c + 1) * CHK, :].astype(
                    jnp.bfloat16)
            for k in range(NC // 2):
                chunk_rdma(R[k], 1).start()
                chunk_rdma(L[k], 3).start()

        q = jnp.dot(
            x_ref[...].astype(jnp.bfloat16),
            wq_ref[...].astype(jnp.bfloat16),
            preferred_element_type=jnp.float32,
        ).astype(jnp.bfloat16)

        @pl.when(my == 1)
        def _():
            for c in R:
                chunk_rdma(c, 0).wait_recv()
                chunk_rdma(c, 2).start()
            for c in L:
                chunk_rdma(c, 0).wait_recv()

        @pl.when(my == 2)
        def _():
            for k in range(NC // 2):
                chunk_rdma(R[k], 0).wait_recv()
                chunk_rdma(R[k], 3).start()
                chunk_rdma(L[k], 0).wait_recv()
                chunk_rdma(L[k], 1).start()

        @pl.when(my == 3)
        def _():
            for c in L:
                chunk_rdma(c, 0).wait_recv()
                chunk_rdma(c, 2).start()
            for c in R:
                chunk_rdma(c, 0).wait_recv()

        kvall = kv_ref[...]
        kk = kvall[:, 0].reshape(SQ, D)
        vv = kvall[:, 1].reshape(SQ, D)

        rb = lax.broadcasted_iota(jnp.int32, (SQ, SQ), 0) // BLK
        cb = lax.broadcasted_iota(jnp.int32, (SQ, SQ), 1) // BLK
        mask = cb <= rb

        outs = []
        for h in range(H):
            hs = slice(h * DH, (h + 1) * DH)
            s = lax.dot_general(
                q[:, hs], kk[:, hs],
                (((1,), (1,)), ((), ())),
                preferred_element_type=jnp.float32,
            ) * SCALE
            s = jnp.where(mask, s, -1e9)
            m = jnp.max(s, axis=1, keepdims=True)
            w = jnp.exp(s - m)
            w = w / jnp.sum(w, axis=1, keepdims=True)
            outs.append(
                lax.dot_general(
                    w.astype(jnp.bfloat16), vv[:, hs],
                    (((1,), (0,)), ((), ())),
                    preferred_element_type=jnp.float32,
                )
            )
        ctx = jnp.concatenate(outs, axis=1).astype(jnp.bfloat16)
        out_ref[...] = jnp.dot(
            ctx, wo_ref[...].astype(jnp.bfloat16),
            preferred_element_type=jnp.float32,
        )

        @pl.when(my == 0)
        def _():
            for c in range(NC):
                chunk_rdma(c, 1).wait_send()

        @pl.when(my == 1)
        def _():
            for c in R:
                chunk_rdma(c, 2).wait_send()

        @pl.when(my == 2)
        def _():
            for c in range(NC):
                chunk_rdma(c, 3).wait_send()

        @pl.when(my == 3)
        def _():
            for c in L:
                chunk_rdma(c, 2).wait_send()

    out2d = pl.pallas_call(
        body,
        out_shape=jax.ShapeDtypeStruct((SQ, D), jnp.float32),
        in_specs=[pl.BlockSpec(memory_space=pltpu.VMEM)] * 5,
        out_specs=pl.BlockSpec(memory_space=pltpu.VMEM),
        scratch_shapes=[
            pltpu.VMEM((NC, 2, CHK, D), jnp.bfloat16),
            pltpu.SemaphoreType.DMA((NC,)),
            pltpu.SemaphoreType.DMA((NC,)),
        ],
        compiler_params=pltpu.CompilerParams(collective_id=0),
    )(xf, Wq, kf, vf, Wo)
    return out2d.reshape(1, SQ, D)


# device time: 48809 ns/iter; 2.4478x vs baseline; 1.2337x over previous
import jax
import jax.numpy as jnp
from jax import lax
from jax.experimental import pallas as pl
from jax.experimental.pallas import tpu as pltpu

N_DEV = 4
SQ = 1024
D = 1024
H = 8
DH = 128
BLK = 64
SCALE = 0.08838834764831843

NC = 8
CHK = SQ // NC

SEND0 = {
    0: [(1, 0)], 2: [(1, 0)],
    1: [(3, 1)], 3: [(3, 1)],
    4: [(1, 0), (3, 1)], 6: [(1, 0), (3, 1)],
    5: [(3, 1), (1, 0)], 7: [(3, 1), (1, 0)],
}
SCHED = {
    1: [(0, 2), (2, 2), (4, 2), (6, 2), (5, None), (7, None),
        (1, None), (3, None)],
    2: [(0, 3), (1, 1), (2, 3), (3, 1), (4, None), (5, None),
        (6, None), (7, None)],
    3: [(1, 2), (3, 2), (5, 2), (7, 2), (4, None), (6, None),
        (0, None), (2, None)],
}


def kernel(x, Wq, K_ext, V_ext, Wo):
    xf = x.reshape(SQ, D)
    kf = K_ext.reshape(K_ext.shape[1], D)
    vf = V_ext.reshape(V_ext.shape[1], D)

    def body(x_ref, wq_ref, k_ref, v_ref, wo_ref, out_ref,
             ctx_ref, send_sems, recv_sems):
        my = lax.axis_index("i")

        barrier = pltpu.get_barrier_semaphore()
        for d in range(N_DEV):
            @pl.when(my != d)
            def _(d=d):
                pl.semaphore_signal(
                    barrier, inc=1,
                    device_id=(d,), device_id_type=pl.DeviceIdType.MESH,
                )
        pl.semaphore_wait(barrier, N_DEV - 1)

        def chunk_rdma(c, target, slot=0):
            return pltpu.make_async_remote_copy(
                src_ref=ctx_ref.at[c], dst_ref=ctx_ref.at[c],
                send_sem=send_sems.at[c, slot], recv_sem=recv_sems.at[c],
                device_id=(target,), device_id_type=pl.DeviceIdType.MESH,
            )

        wob = wo_ref[...].astype(jnp.bfloat16)

        def wo_chunk(c):
            out_ref[c * CHK:(c + 1) * CHK, :] = jnp.dot(
                ctx_ref[c], wob, preferred_element_type=jnp.float32)

        @pl.when(my == 0)
        def _():
            kkb = k_ref[...].astype(jnp.bfloat16)
            vvb = v_ref[...].astype(jnp.bfloat16)
            q = jnp.dot(
                x_ref[...].astype(jnp.bfloat16),
                wq_ref[...].astype(jnp.bfloat16),
                preferred_element_type=jnp.float32,
            ).astype(jnp.bfloat16)
            for c in range(NC):
                nk = CHK * (c + 1)
                qc = q[c * CHK:(c + 1) * CHK, :]
                rb = (lax.broadcasted_iota(jnp.int32, (CHK, nk), 0)
                      + c * CHK) // BLK
                cb = lax.broadcasted_iota(jnp.int32, (CHK, nk), 1) // BLK
                msk = cb <= rb
                for h in range(H):
                    hs = slice(h * DH, (h + 1) * DH)
                    s = lax.dot_general(
                        qc[:, hs], kkb[:nk, hs],
                        (((1,), (1,)), ((), ())),
                        preferred_element_type=jnp.float32,
                    ) * SCALE
                    s = jnp.where(msk, s, -1e9)
                    m = jnp.max(s, axis=1, keepdims=True)
                    w = jnp.exp(s - m)
                    w = w / jnp.sum(w, axis=1, keepdims=True)
                    ctx_ref[c, :, hs] = lax.dot_general(
                        w.astype(jnp.bfloat16), vvb[:nk, hs],
                        (((1,), (0,)), ((), ())),
                        preferred_element_type=jnp.float32,
                    ).astype(jnp.bfloat16)
                for tgt, slot in SEND0[c]:
                    chunk_rdma(c, tgt, slot).start()
            for c in range(NC):
                wo_chunk(c)
            for c, routes in SEND0.items():
                for _, slot in routes:
                    chunk_rdma(c, 1, slot).wait_send()

        for dev, sched in SCHED.items():
            @pl.when(my == dev)
            def _(sched=sched):
                for c, fwd in sched:
                    chunk_rdma(c, 0).wait_recv()
                    if fwd is not None:
                        chunk_rdma(c, fwd).start()
                    wo_chunk(c)
                for c, fwd in sched:
                    if fwd is not None:
                        chunk_rdma(c, fwd).wait_send()

    out2d = pl.pallas_call(
        body,
        out_shape=jax.ShapeDtypeStruct((SQ, D), jnp.float32),
        in_specs=[pl.BlockSpec(memory_space=pltpu.VMEM)] * 5,
        out_specs=pl.BlockSpec(memory_space=pltpu.VMEM),
        scratch_shapes=[
            pltpu.VMEM((NC, CHK, D), jnp.bfloat16),
            pltpu.SemaphoreType.DMA((NC, 2)),
            pltpu.SemaphoreType.DMA((NC,)),
        ],
        compiler_params=pltpu.CompilerParams(collective_id=0),
    )(xf, Wq, kf, vf, Wo)
    return out2d.reshape(1, SQ, D)


# device time: 40198 ns/iter; 2.9722x vs baseline; 1.2142x over previous
import jax
import jax.numpy as jnp
from jax import lax
from jax.experimental import pallas as pl
from jax.experimental.pallas import tpu as pltpu

N_DEV = 4
SQ = 1024
D = 1024
H = 8
DH = 128
BLK = 64
SCALE = 0.08838834764831843

NC = 8
CHK = SQ // NC

SEND0 = {
    0: [(1, 0)], 2: [(1, 0)],
    1: [(3, 1)], 3: [(3, 1)],
    4: [(1, 0), (3, 1)], 6: [(1, 0), (3, 1)],
    5: [(3, 1), (1, 0)], 7: [(3, 1), (1, 0)],
}
SCHED = {
    1: [(0, 2), (2, 2), (4, 2), (6, 2), (5, None), (7, None),
        (1, None), (3, None)],
    2: [(0, 3), (1, 1), (2, 3), (3, 1), (4, None), (5, None),
        (6, None), (7, None)],
    3: [(1, 2), (3, 2), (5, 2), (7, 2), (4, None), (6, None),
        (0, None), (2, None)],
}


def kernel(x, Wq, K_ext, V_ext, Wo):
    xf = x.reshape(SQ, D)
    kf = K_ext.reshape(K_ext.shape[1], D)
    vf = V_ext.reshape(V_ext.shape[1], D)

    def body(x_ref, wq_ref, k_ref, v_ref, wo_ref, out_ref,
             ctx_ref, send_sems, recv_sems):
        my = lax.axis_index("i")

        barrier = pltpu.get_barrier_semaphore()
        for d in range(N_DEV):
            @pl.when(my != d)
            def _(d=d):
                pl.semaphore_signal(
                    barrier, inc=1,
                    device_id=(d,), device_id_type=pl.DeviceIdType.MESH,
                )
        pl.semaphore_wait(barrier, N_DEV - 1)

        def chunk_rdma(c, target, slot=0):
            return pltpu.make_async_remote_copy(
                src_ref=ctx_ref.at[c], dst_ref=ctx_ref.at[c],
                send_sem=send_sems.at[c, slot], recv_sem=recv_sems.at[c],
                device_id=(target,), device_id_type=pl.DeviceIdType.MESH,
            )

        wob = wo_ref[...].astype(jnp.bfloat16)

        def wo_chunk(c):
            out_ref[c * CHK:(c + 1) * CHK, :] = jnp.dot(
                ctx_ref[c], wob, preferred_element_type=jnp.float32)

        @pl.when(my == 0)
        def _():
            kkb = k_ref[...].astype(jnp.bfloat16)
            vvb = v_ref[...].astype(jnp.bfloat16)
            xb = x_ref[...].astype(jnp.bfloat16)
            wqb = wq_ref[...].astype(jnp.bfloat16)

            def qproj(lo, hi):
                return (jnp.dot(xb[lo:hi, :], wqb,
                                preferred_element_type=jnp.float32)
                        * SCALE).astype(jnp.bfloat16)

            q = qproj(0, SQ // 2)
            for c in range(NC):
                if c == NC // 2:
                    q = qproj(SQ // 2, SQ)
                nk = CHK * (c + 1)
                qc = q[(c % (NC // 2)) * CHK:(c % (NC // 2) + 1) * CHK, :]
                rb = (lax.broadcasted_iota(jnp.int32, (CHK, nk), 0)
                      + c * CHK) // BLK
                cb = lax.broadcasted_iota(jnp.int32, (CHK, nk), 1) // BLK
                msk = cb <= rb
                for h in range(H):
                    hs = slice(h * DH, (h + 1) * DH)
                    s = lax.dot_general(
                        qc[:, hs], kkb[:nk, hs],
                        (((1,), (1,)), ((), ())),
                        preferred_element_type=jnp.float32,
                    )
                    w = jnp.where(msk, jnp.exp(s), 0.0)
                    l = jnp.sum(w, axis=1, keepdims=True)
                    ctx_ref[c, :, hs] = (lax.dot_general(
                        w.astype(jnp.bfloat16), vvb[:nk, hs],
                        (((1,), (0,)), ((), ())),
                        preferred_element_type=jnp.float32,
                    ) / l).astype(jnp.bfloat16)
                for tgt, slot in SEND0[c]:
                    chunk_rdma(c, tgt, slot).start()
            for c in range(NC):
                wo_chunk(c)
            for c, routes in SEND0.items():
                for _, slot in routes:
                    chunk_rdma(c, 1, slot).wait_send()

        for dev, sched in SCHED.items():
            @pl.when(my == dev)
            def _(sched=sched):
                for c, fwd in sched:
                    chunk_rdma(c, 0).wait_recv()
                    if fwd is not None:
                        chunk_rdma(c, fwd).start()
                    wo_chunk(c)
                for c, fwd in sched:
                    if fwd is not None:
                        chunk_rdma(c, fwd).wait_send()

    out2d = pl.pallas_call(
        body,
        out_shape=jax.ShapeDtypeStruct((SQ, D), jnp.float32),
        in_specs=[pl.BlockSpec(memory_space=pltpu.VMEM)] * 5,
        out_specs=pl.BlockSpec(memory_space=pltpu.VMEM),
        scratch_shapes=[
            pltpu.VMEM((NC, CHK, D), jnp.bfloat16),
            pltpu.SemaphoreType.DMA((NC, 2)),
            pltpu.SemaphoreType.DMA((NC,)),
        ],
        compiler_params=pltpu.CompilerParams(collective_id=0),
    )(xf, Wq, kf, vf, Wo)
    return out2d.reshape(1, SQ, D)
